# Initial kernel scaffold; baseline (speedup 1.0000x reference)
import jax
import jax.numpy as jnp
from jax import lax
from jax.experimental import pallas as pl
from jax.experimental.pallas import tpu as pltpu
from jax.experimental.pallas import tpu_sc as plsc

N = 10000
E = 320000
D = 128
HID = 128
OUT = 10
G = 64

NC = 2
NS = 16
NW = NC * NS

CHUNK = 128
CHUNKS_PER_W = 79
E_PAD = NW * CHUNKS_PER_W * CHUNK
DUMMY = N
AGG_ROWS = 10016
ROWS_PER_TILE = AGG_ROWS // NS


def _sc_agg_body(x_hbm, src_hbm, dst_hbm, zeros_hbm, out_hbm,
                 sidx_v, didx_v, rows_v, sem):
    c = lax.axis_index("c")
    s = lax.axis_index("s")
    wid = c * NS + s
    r0 = s * ROWS_PER_TILE

    def scoped(agg_sh):
        pltpu.sync_copy(zeros_hbm.at[pl.ds(r0, ROWS_PER_TILE)],
                        agg_sh.at[pl.ds(r0, ROWS_PER_TILE)])
        plsc.subcore_barrier()

        base = wid * CHUNKS_PER_W * CHUNK

        def body(i, carry):
            e0 = base + i * CHUNK
            pltpu.sync_copy(src_hbm.at[pl.ds(e0, CHUNK)], sidx_v)
            pltpu.sync_copy(dst_hbm.at[pl.ds(e0, CHUNK)], didx_v)
            pltpu.async_copy(x_hbm.at[sidx_v], rows_v, sem).wait()
            pltpu.sync_copy(rows_v, agg_sh.at[didx_v], add=True)
            return carry

        lax.fori_loop(0, CHUNKS_PER_W, body, 0)
        plsc.subcore_barrier()

        pltpu.sync_copy(agg_sh.at[pl.ds(r0, ROWS_PER_TILE)],
                        out_hbm.at[c].at[pl.ds(r0, ROWS_PER_TILE)])

    pl.run_scoped(scoped, pltpu.VMEM_SHARED((AGG_ROWS, D), jnp.float32))


_sc_agg = pl.kernel(
    _sc_agg_body,
    out_type=jax.ShapeDtypeStruct((NC, AGG_ROWS, D), jnp.float32),
    mesh=plsc.VectorSubcoreMesh(core_axis_name="c", subcore_axis_name="s",
                                num_cores=NC, num_subcores=NS),
    scratch_types=[
        pltpu.VMEM((CHUNK,), jnp.int32),
        pltpu.VMEM((CHUNK,), jnp.int32),
        pltpu.VMEM((CHUNK, D), jnp.float32),
        pltpu.SemaphoreType.DMA,
    ],
)


def _dense1_body(aggp_ref, x_ref, wrel_ref, b_ref, wroot_ref, g_ref, be_ref,
                 out_ref):
    agg = aggp_ref[0, :N, :] + aggp_ref[1, :N, :]
    x = x_ref[...]
    y = lax.dot_general(agg, wrel_ref[...], (((1,), (1,)), ((), ())),
                        preferred_element_type=jnp.float32)
    y += lax.dot_general(x, wroot_ref[...], (((1,), (1,)), ((), ())),
                         preferred_element_type=jnp.float32)
    y += b_ref[...][None, :]
    mean = jnp.mean(y, axis=0, keepdims=True)
    var = jnp.mean((y - mean) * (y - mean), axis=0, keepdims=True)
    yn = (y - mean) * lax.rsqrt(var + 1e-5)
    yn = yn * g_ref[...][None, :] + be_ref[...][None, :]
    out_ref[...] = jnp.maximum(yn, 0.0)


_dense1 = pl.pallas_call(
    _dense1_body,
    out_shape=jax.ShapeDtypeStruct((N, HID), jnp.float32),
)


def _dense2_body(aggp_ref, h_ref, batch_ref, wrel_ref, b_ref, wroot_ref,
                 g_ref, be_ref, wc_ref, bc_ref, logits_ref, h2_ref):
    agg = aggp_ref[0, :N, :] + aggp_ref[1, :N, :]
    h = h_ref[...]
    y = lax.dot_general(agg, wrel_ref[...], (((1,), (1,)), ((), ())),
                        preferred_element_type=jnp.float32)
    y += lax.dot_general(h, wroot_ref[...], (((1,), (1,)), ((), ())),
                         preferred_element_type=jnp.float32)
    y += b_ref[...][None, :]
    mean = jnp.mean(y, axis=0, keepdims=True)
    var = jnp.mean((y - mean) * (y - mean), axis=0, keepdims=True)
    yn = (y - mean) * lax.rsqrt(var + 1e-5)
    yn = yn * g_ref[...][None, :] + be_ref[...][None, :]
    h2 = jnp.maximum(yn, 0.0)
    h2_ref[...] = h2

    gids = lax.broadcasted_iota(jnp.int32, (G, N), 0)
    onehot = jnp.where(batch_ref[...] == gids, 1.0, 0.0)
    pooled = lax.dot_general(onehot, h2, (((1,), (0,)), ((), ())),
                             preferred_element_type=jnp.float32)
    logits = lax.dot_general(pooled, wc_ref[...], (((1,), (1,)), ((), ())),
                             preferred_element_type=jnp.float32)
    logits_ref[...] = logits + bc_ref[...][None, :]


_dense2 = pl.pallas_call(
    _dense2_body,
    out_shape=[
        jax.ShapeDtypeStruct((G, OUT), jnp.float32),
        jax.ShapeDtypeStruct((N, HID), jnp.float32),
    ],
)


@jax.jit
def kernel(x, edge_index, batch, W1_rel, b1, W1_root, g1, be1,
           W2_rel, b2, W2_root, g2, be2, Wc, bc):
    src = edge_index[0]
    dst = edge_index[1]
    pad = E_PAD - E
    src_p = jnp.concatenate([src, jnp.zeros((pad,), jnp.int32)])
    dst_p = jnp.concatenate([dst, jnp.full((pad,), DUMMY, jnp.int32)])
    zeros = jnp.zeros((AGG_ROWS, D), jnp.float32)

    aggp1 = _sc_agg(x, src_p, dst_p, zeros)
    h1 = _dense1(aggp1, x, W1_rel, b1, W1_root, g1, be1)
    aggp2 = _sc_agg(h1, src_p, dst_p, zeros)
    logits, h2 = _dense2(aggp2, h1, batch.reshape(1, N), W2_rel, b2,
                         W2_root, g2, be2, Wc, bc)
    return (logits, h2)


# speedup vs baseline: 4.0105x; 4.0105x over previous
import jax
import jax.numpy as jnp
from jax import lax
from jax.experimental import pallas as pl
from jax.experimental.pallas import tpu as pltpu
from jax.experimental.pallas import tpu_sc as plsc

N = 10000
E = 320000
D = 128
HID = 128
OUT = 10
G = 64

NC = 2
NS = 16
NW = NC * NS

CHUNK = 128
CHUNKS_PER_W = 79
E_PAD = NW * CHUNKS_PER_W * CHUNK
DUMMY = N
AGG_ROWS = 10112
ROWS_PER_TILE = AGG_ROWS // NS


def _sc_agg_body(x_hbm, src_hbm, dst_hbm, zeros_hbm, out_hbm,
                 sidx_v, didx_v, rows_v, agg_sh, sem):
    c = lax.axis_index("c")
    s = lax.axis_index("s")
    wid = c * NS + s
    r0 = s * ROWS_PER_TILE

    pltpu.sync_copy(zeros_hbm.at[pl.ds(r0, ROWS_PER_TILE)],
                    agg_sh.at[pl.ds(r0, ROWS_PER_TILE)])
    plsc.subcore_barrier()

    base = wid * CHUNKS_PER_W * CHUNK

    def body(i, carry):
        e0 = base + i * CHUNK
        pltpu.sync_copy(src_hbm.at[pl.ds(e0, CHUNK)], sidx_v)
        pltpu.sync_copy(dst_hbm.at[pl.ds(e0, CHUNK)], didx_v)
        pltpu.async_copy(x_hbm.at[sidx_v], rows_v, sem).wait()
        pltpu.sync_copy(rows_v, agg_sh.at[didx_v], add=True)
        return carry

    lax.fori_loop(0, CHUNKS_PER_W, body, 0)
    plsc.subcore_barrier()

    pltpu.sync_copy(agg_sh.at[pl.ds(r0, ROWS_PER_TILE)],
                    out_hbm.at[c].at[pl.ds(r0, ROWS_PER_TILE)])


_sc_agg = pl.kernel(
    _sc_agg_body,
    out_type=jax.ShapeDtypeStruct((NC, AGG_ROWS, D), jnp.float32),
    mesh=plsc.VectorSubcoreMesh(core_axis_name="c", subcore_axis_name="s",
                                num_cores=NC, num_subcores=NS),
    scratch_types=[
        pltpu.VMEM((CHUNK,), jnp.int32),
        pltpu.VMEM((CHUNK,), jnp.int32),
        pltpu.VMEM((CHUNK, D), jnp.float32),
        pltpu.VMEM_SHARED((AGG_ROWS, D), jnp.float32),
        pltpu.SemaphoreType.DMA,
    ],
)


def _dense1_body(aggp_ref, x_ref, wrel_ref, b_ref, wroot_ref, g_ref, be_ref,
                 out_ref):
    agg = aggp_ref[0, :N, :] + aggp_ref[1, :N, :]
    x = x_ref[...]
    y = lax.dot_general(agg, wrel_ref[...], (((1,), (1,)), ((), ())),
                        preferred_element_type=jnp.float32)
    y += lax.dot_general(x, wroot_ref[...], (((1,), (1,)), ((), ())),
                         preferred_element_type=jnp.float32)
    y += b_ref[...][None, :]
    mean = jnp.mean(y, axis=0, keepdims=True)
    var = jnp.mean((y - mean) * (y - mean), axis=0, keepdims=True)
    yn = (y - mean) * lax.rsqrt(var + 1e-5)
    yn = yn * g_ref[...][None, :] + be_ref[...][None, :]
    out_ref[...] = jnp.maximum(yn, 0.0)


_dense1 = pl.pallas_call(
    _dense1_body,
    out_shape=jax.ShapeDtypeStruct((N, HID), jnp.float32),
)


def _dense2_body(aggp_ref, h_ref, batch_ref, wrel_ref, b_ref, wroot_ref,
                 g_ref, be_ref, wc_ref, bc_ref, logits_ref, h2_ref):
    agg = aggp_ref[0, :N, :] + aggp_ref[1, :N, :]
    h = h_ref[...]
    y = lax.dot_general(agg, wrel_ref[...], (((1,), (1,)), ((), ())),
                        preferred_element_type=jnp.float32)
    y += lax.dot_general(h, wroot_ref[...], (((1,), (1,)), ((), ())),
                         preferred_element_type=jnp.float32)
    y += b_ref[...][None, :]
    mean = jnp.mean(y, axis=0, keepdims=True)
    var = jnp.mean((y - mean) * (y - mean), axis=0, keepdims=True)
    yn = (y - mean) * lax.rsqrt(var + 1e-5)
    yn = yn * g_ref[...][None, :] + be_ref[...][None, :]
    h2 = jnp.maximum(yn, 0.0)
    h2_ref[...] = h2

    gids = lax.broadcasted_iota(jnp.int32, (G, N), 0)
    onehot = jnp.where(batch_ref[...] == gids, 1.0, 0.0)
    pooled = lax.dot_general(onehot, h2, (((1,), (0,)), ((), ())),
                             preferred_element_type=jnp.float32)
    logits = lax.dot_general(pooled, wc_ref[...], (((1,), (1,)), ((), ())),
                             preferred_element_type=jnp.float32)
    logits_ref[...] = logits + bc_ref[...][None, :]


_dense2 = pl.pallas_call(
    _dense2_body,
    out_shape=[
        jax.ShapeDtypeStruct((G, OUT), jnp.float32),
        jax.ShapeDtypeStruct((N, HID), jnp.float32),
    ],
)


@jax.jit
def kernel(x, edge_index, batch, W1_rel, b1, W1_root, g1, be1,
           W2_rel, b2, W2_root, g2, be2, Wc, bc):
    src = edge_index[0]
    dst = edge_index[1]
    pad = E_PAD - E
    src_p = jnp.concatenate([src, jnp.zeros((pad,), jnp.int32)])
    dst_p = jnp.concatenate([dst, jnp.full((pad,), DUMMY, jnp.int32)])
    zeros = jnp.zeros((AGG_ROWS, D), jnp.float32)

    aggp1 = _sc_agg(x, src_p, dst_p, zeros)
    h1 = _dense1(aggp1, x, W1_rel, b1, W1_root, g1, be1)
    aggp2 = _sc_agg(h1, src_p, dst_p, zeros)
    logits, h2 = _dense2(aggp2, h1, batch.reshape(1, N), W2_rel, b2,
                         W2_root, g2, be2, Wc, bc)
    return (logits, h2)
